# baseline (speedup 1.0000x reference)
import functools

import jax
import jax.numpy as jnp
from jax import lax
from jax.experimental import pallas as pl
from jax.experimental.pallas import tpu as pltpu
from jax.experimental.pallas import tpu_sc as plsc

_TARGET_RATE = 0.75
_TEMPERATURE = 0.5

_NC, _NS, _L = 2, 16, 16


def _sc_mesh():
    return plsc.VectorSubcoreMesh(
        core_axis_name="c", subcore_axis_name="s",
        num_cores=_NC, num_subcores=_NS)


def _mask_sc_body(imp_hbm, g2_hbm, tr_hbm, mask_hbm,
                  imp_v, g2_v, tr_v, out_v, hist, *, k, B, HW):
    nch = HW // _L
    w = lax.axis_index("s") * _NC + lax.axis_index("c")

    @pl.when(w < B)
    def _():
        pltpu.sync_copy(imp_hbm.at[w], imp_v)
        pltpu.sync_copy(g2_hbm.at[w], g2_v)
        pltpu.sync_copy(tr_hbm, tr_v)

        lanes = lax.iota(jnp.int32, _L)
        ones = jnp.ones((_L,), jnp.int32)
        zeros16 = jnp.zeros((_L,), jnp.int32)

        prefix = jnp.int32(0)
        k_rem = jnp.int32(k)
        for p in range(8):
            s = 28 - 4 * p
            sh = min(s + 4, 31)
            for z in range(_L):
                hist[pl.ds(z * _L, _L)] = zeros16

            def scan_chunk(c, acc, _s=s, _sh=sh, _prefix=prefix):
                for uu in range(4):
                    v = imp_v[pl.ds((c * 4 + uu) * _L, _L)]
                    bits = lax.bitcast_convert_type(v, jnp.int32)
                    match = lax.shift_right_logical(bits, _sh) == _prefix
                    digit = lax.shift_right_logical(bits, _s) & 15
                    plsc.addupdate_scatter(hist, [lanes * _L + digit], ones,
                                           mask=match)
                return acc

            lax.fori_loop(0, nch // 4, scan_chunk, jnp.int32(0))

            cnt = jnp.zeros((_L,), jnp.int32)
            for z in range(_L):
                cnt = cnt + hist[pl.ds(z * _L, _L)]
            rev = lax.rev(cnt, (0,))
            csrev = plsc.cumsum(rev)
            sfx_ex = csrev - rev
            is_sel = (csrev >= k_rem) & (sfx_ex < k_rem)
            d_star = jnp.sum(jnp.where(is_sel, 15 - lanes, 0))
            cnt_sel = jnp.sum(jnp.where(is_sel, rev, 0))
            k_rem = k_rem - jnp.sum(jnp.where(is_sel, sfx_ex, 0))
            prefix = lax.shift_left(prefix, 4) | d_star

        m = prefix
        extra = k_rem

        sel_train = tr_v[...] != 0

        @pl.when(cnt_sel == extra)
        def _no_excess_ties():
            def fin_fast(c, acc):
                for uu in range(4):
                    off = pl.ds((c * 4 + uu) * _L, _L)
                    v = imp_v[off]
                    bits = lax.bitcast_convert_type(v, jnp.int32)
                    take = bits >= m
                    train = v > g2_v[off] * (1.0 - v + 1e-07)
                    fin = jnp.where(sel_train, train, take)
                    out_v[off] = jnp.where(fin, 1.0, 0.0)
                return acc

            lax.fori_loop(0, nch // 4, fin_fast, jnp.int32(0))

        @pl.when(cnt_sel != extra)
        def _excess_ties():
            def fin_chunk(c, run_eq):
                v = imp_v[pl.ds(c * _L, _L)]
                bits = lax.bitcast_convert_type(v, jnp.int32)
                gt = bits > m
                eq = bits == m
                eqi = eq.astype(jnp.int32)
                ceq = plsc.cumsum(eqi) + run_eq
                take = gt | (eq & (ceq <= extra))
                train = v > g2_v[pl.ds(c * _L, _L)] * (1.0 - v + 1e-07)
                fin = jnp.where(sel_train, train, take)
                out_v[pl.ds(c * _L, _L)] = jnp.where(fin, 1.0, 0.0)
                return run_eq + jnp.sum(eqi)

            lax.fori_loop(0, nch, fin_chunk, jnp.int32(0))

        pltpu.sync_copy(out_v, mask_hbm.at[w])


def _mask_sc(imp2d, g2_2d, tr16, *, k, B, HW):
    body = functools.partial(_mask_sc_body, k=k, B=B, HW=HW)
    return pl.kernel(
        body,
        out_type=jax.ShapeDtypeStruct((B, HW), jnp.float32),
        mesh=_sc_mesh(),
        scratch_types=[
            pltpu.VMEM((HW,), jnp.float32),
            pltpu.VMEM((HW,), jnp.float32),
            pltpu.VMEM((_L,), jnp.int32),
            pltpu.VMEM((HW,), jnp.float32),
            pltpu.VMEM((_L * _L,), jnp.int32),
        ],
        compiler_params=pltpu.CompilerParams(needs_layout_passes=False),
    )(imp2d, g2_2d, tr16)


def _mems_sc_body(mm_hbm, mask_hbm, out_hbm,
                  mv_, bin0, bin1, bout0, bout1,
                  si0, si1, so0, so1, *, T, B, H, W, C, RPH):
    nchunks = H // RPH
    nc16 = C // _L
    wk = lax.axis_index("s") * _NC + lax.axis_index("c")
    t = lax.div(wk, B)
    b = lax.rem(wk, B)
    pltpu.sync_copy(mask_hbm.at[b], mv_)

    bins = (bin0, bin1)
    bouts = (bout0, bout1)
    sis = (si0, si1)
    sos = (so0, so1)

    in_cps = [None] * nchunks
    out_cps = [None] * nchunks
    in_cps[0] = pltpu.async_copy(mm_hbm.at[t, b, pl.ds(0, RPH)], bins[0],
                                 sis[0])
    for g in range(nchunks):
        pb = g % 2
        if g + 1 < nchunks:
            in_cps[g + 1] = pltpu.async_copy(
                mm_hbm.at[t, b, pl.ds((g + 1) * RPH, RPH)],
                bins[(g + 1) % 2], sis[(g + 1) % 2])
        in_cps[g].wait()
        if g >= 2:
            out_cps[g - 2].wait()
        bi, bo = bins[pb], bouts[pb]

        def mul_w(wj, acc, _bi=bi, _bo=bo, _g=g):
            for wu in range(2):
                wi = wj * 2 + wu
                iw = jnp.full((_L,), wi, jnp.int32)
                for r in range(RPH):
                    ir = jnp.full((_L,), _g * RPH + r, jnp.int32)
                    mvec = plsc.load_gather(mv_, [ir, iw])
                    for c6 in range(nc16):
                        off = pl.ds(c6 * _L, _L)
                        _bo[r, wi, off] = _bi[r, wi, off] * mvec
            return acc

        lax.fori_loop(0, W // 2, mul_w, jnp.int32(0))
        out_cps[g] = pltpu.async_copy(
            bo, out_hbm.at[t, b, pl.ds(g * RPH, RPH)], sos[pb])
    out_cps[nchunks - 2].wait()
    out_cps[nchunks - 1].wait()


def _mems_sc(mmT, maskT, *, RPH):
    T, B, H, W, C = mmT.shape
    body = functools.partial(_mems_sc_body, T=T, B=B, H=H, W=W, C=C, RPH=RPH)
    return pl.kernel(
        body,
        out_type=jax.ShapeDtypeStruct((T, B, H, W, C), jnp.float32),
        mesh=_sc_mesh(),
        scratch_types=[
            pltpu.VMEM((H, W), jnp.float32),
            pltpu.VMEM((RPH, W, C), jnp.float32),
            pltpu.VMEM((RPH, W, C), jnp.float32),
            pltpu.VMEM((RPH, W, C), jnp.float32),
            pltpu.VMEM((RPH, W, C), jnp.float32),
            pltpu.SemaphoreType.DMA,
            pltpu.SemaphoreType.DMA,
            pltpu.SemaphoreType.DMA,
            pltpu.SemaphoreType.DMA,
        ],
        compiler_params=pltpu.CompilerParams(needs_layout_passes=False),
    )(mmT, maskT)


def _spikes_tc_body(mask_ref, sp_ref, os_ref):
    os_ref[...] = sp_ref[...] * mask_ref[...][None, :, :, :, None]


def _spikes_tc(spT, maskT, *, BH):
    T, B, H, W, C = spT.shape
    return pl.pallas_call(
        _spikes_tc_body,
        grid=(T, H // BH),
        in_specs=[
            pl.BlockSpec((B, BH, W), lambda i, j: (0, j, 0)),
            pl.BlockSpec((1, B, BH, W, C), lambda i, j: (i, 0, j, 0, 0)),
        ],
        out_specs=pl.BlockSpec((1, B, BH, W, C), lambda i, j: (i, 0, j, 0, 0)),
        out_shape=jax.ShapeDtypeStruct((T, B, H, W, C), jnp.float32),
    )(maskT, spT)


def kernel(spikes, mems, importance, training=0, target_override=None):
    target = target_override if target_override is not None else _TARGET_RATE
    T, B, C, H, W = spikes.shape
    HW = H * W
    k = max(1, int(target * HW))

    imp2d = importance.reshape(B, HW)
    u = jnp.clip(
        jax.random.uniform(jax.random.key(42), (B, 1, H, W), dtype=jnp.float32),
        1e-07, 1.0 - 1e-07).reshape(B, HW)
    g2_2d = -jnp.log(u) - 1e-07
    tr16 = jnp.broadcast_to(jnp.asarray(training, jnp.int32), (_L,))

    mask2d = _mask_sc(imp2d, g2_2d, tr16, k=k, B=B, HW=HW)
    maskT = mask2d.reshape(B, H, W)

    spT = jnp.transpose(spikes, (0, 1, 3, 4, 2))
    mmT = jnp.transpose(mems, (0, 1, 3, 4, 2))

    os_ = _spikes_tc(spT, maskT, BH=H)
    om_ = _mems_sc(mmT, maskT, RPH=2)

    return (jnp.transpose(os_, (0, 1, 4, 2, 3)),
            jnp.transpose(om_, (0, 1, 4, 2, 3)),
            mask2d.reshape(B, 1, H, W))


# speedup vs baseline: 1.0352x; 1.0352x over previous
import functools

import jax
import jax.numpy as jnp
from jax import lax
from jax.experimental import pallas as pl
from jax.experimental.pallas import tpu as pltpu
from jax.experimental.pallas import tpu_sc as plsc

_TARGET_RATE = 0.75
_TEMPERATURE = 0.5

_NC, _NS, _L = 2, 16, 16


def _sc_mesh():
    return plsc.VectorSubcoreMesh(
        core_axis_name="c", subcore_axis_name="s",
        num_cores=_NC, num_subcores=_NS)


def _mask_sc_body(imp_hbm, g2_hbm, tr_hbm, mask_hbm,
                  imp_v, g2_v, tr_v, out_v, hist, *, k, B, HW):
    nch = HW // _L
    w = lax.axis_index("s") * _NC + lax.axis_index("c")

    @pl.when(w < B)
    def _():
        pltpu.sync_copy(imp_hbm.at[w], imp_v)
        pltpu.sync_copy(g2_hbm.at[w], g2_v)
        pltpu.sync_copy(tr_hbm, tr_v)

        lanes = lax.iota(jnp.int32, _L)
        ones = jnp.ones((_L,), jnp.int32)
        zeros16 = jnp.zeros((_L,), jnp.int32)

        prefix = jnp.int32(0)
        k_rem = jnp.int32(k)
        for p in range(8):
            s = 28 - 4 * p
            sh = min(s + 4, 31)
            for z in range(_L):
                hist[pl.ds(z * _L, _L)] = zeros16

            def scan_chunk(c, acc, _s=s, _sh=sh, _prefix=prefix):
                for uu in range(4):
                    v = imp_v[pl.ds((c * 4 + uu) * _L, _L)]
                    bits = lax.bitcast_convert_type(v, jnp.int32)
                    match = lax.shift_right_logical(bits, _sh) == _prefix
                    digit = lax.shift_right_logical(bits, _s) & 15
                    plsc.addupdate_scatter(hist, [lanes * _L + digit], ones,
                                           mask=match)
                return acc

            lax.fori_loop(0, nch // 4, scan_chunk, jnp.int32(0))

            cnt = jnp.zeros((_L,), jnp.int32)
            for z in range(_L):
                cnt = cnt + hist[pl.ds(z * _L, _L)]
            rev = lax.rev(cnt, (0,))
            csrev = plsc.cumsum(rev)
            sfx_ex = csrev - rev
            is_sel = (csrev >= k_rem) & (sfx_ex < k_rem)
            d_star = jnp.sum(jnp.where(is_sel, 15 - lanes, 0))
            cnt_sel = jnp.sum(jnp.where(is_sel, rev, 0))
            k_rem = k_rem - jnp.sum(jnp.where(is_sel, sfx_ex, 0))
            prefix = lax.shift_left(prefix, 4) | d_star

        m = prefix
        extra = k_rem

        sel_train = tr_v[...] != 0

        @pl.when(cnt_sel == extra)
        def _no_excess_ties():
            def fin_fast(c, acc):
                for uu in range(4):
                    off = pl.ds((c * 4 + uu) * _L, _L)
                    v = imp_v[off]
                    bits = lax.bitcast_convert_type(v, jnp.int32)
                    take = bits >= m
                    train = v > g2_v[off] * (1.0 - v + 1e-07)
                    fin = jnp.where(sel_train, train, take)
                    out_v[off] = jnp.where(fin, 1.0, 0.0)
                return acc

            lax.fori_loop(0, nch // 4, fin_fast, jnp.int32(0))

        @pl.when(cnt_sel != extra)
        def _excess_ties():
            def fin_chunk(c, run_eq):
                v = imp_v[pl.ds(c * _L, _L)]
                bits = lax.bitcast_convert_type(v, jnp.int32)
                gt = bits > m
                eq = bits == m
                eqi = eq.astype(jnp.int32)
                ceq = plsc.cumsum(eqi) + run_eq
                take = gt | (eq & (ceq <= extra))
                train = v > g2_v[pl.ds(c * _L, _L)] * (1.0 - v + 1e-07)
                fin = jnp.where(sel_train, train, take)
                out_v[pl.ds(c * _L, _L)] = jnp.where(fin, 1.0, 0.0)
                return run_eq + jnp.sum(eqi)

            lax.fori_loop(0, nch, fin_chunk, jnp.int32(0))

        pltpu.sync_copy(out_v, mask_hbm.at[w])


def _mask_sc(imp2d, g2_2d, tr16, *, k, B, HW):
    body = functools.partial(_mask_sc_body, k=k, B=B, HW=HW)
    return pl.kernel(
        body,
        out_type=jax.ShapeDtypeStruct((B, HW), jnp.float32),
        mesh=_sc_mesh(),
        scratch_types=[
            pltpu.VMEM((HW,), jnp.float32),
            pltpu.VMEM((HW,), jnp.float32),
            pltpu.VMEM((_L,), jnp.int32),
            pltpu.VMEM((HW,), jnp.float32),
            pltpu.VMEM((_L * _L,), jnp.int32),
        ],
        compiler_params=pltpu.CompilerParams(needs_layout_passes=False),
    )(imp2d, g2_2d, tr16)


def _mems_sc_body(mm_hbm, mask_hbm, out_hbm,
                  mv_, bin0, bin1, bout0, bout1,
                  si0, si1, so0, so1, *, T, B, H, W, C, RPH):
    nchunks = H // RPH
    nc16 = C // _L
    wk = lax.axis_index("s") * _NC + lax.axis_index("c")
    t = lax.div(wk, B)
    b = lax.rem(wk, B)
    pltpu.sync_copy(mask_hbm.at[b], mv_)

    bins = (bin0, bin1)
    bouts = (bout0, bout1)
    sis = (si0, si1)
    sos = (so0, so1)

    in_cps = [None] * nchunks
    out_cps = [None] * nchunks
    in_cps[0] = pltpu.async_copy(mm_hbm.at[t, b, pl.ds(0, RPH)], bins[0],
                                 sis[0])
    for g in range(nchunks):
        pb = g % 2
        if g + 1 < nchunks:
            in_cps[g + 1] = pltpu.async_copy(
                mm_hbm.at[t, b, pl.ds((g + 1) * RPH, RPH)],
                bins[(g + 1) % 2], sis[(g + 1) % 2])
        in_cps[g].wait()
        if g >= 2:
            out_cps[g - 2].wait()
        bi, bo = bins[pb], bouts[pb]

        def mul_w(wj, acc, _bi=bi, _bo=bo, _g=g):
            for wu in range(2):
                wi = wj * 2 + wu
                iw = jnp.full((_L,), wi, jnp.int32)
                for r in range(RPH):
                    ir = jnp.full((_L,), _g * RPH + r, jnp.int32)
                    mvec = plsc.load_gather(mv_, [ir, iw])
                    for c6 in range(nc16):
                        off = pl.ds(c6 * _L, _L)
                        _bo[r, wi, off] = _bi[r, wi, off] * mvec
            return acc

        lax.fori_loop(0, W // 2, mul_w, jnp.int32(0))
        out_cps[g] = pltpu.async_copy(
            bo, out_hbm.at[t, b, pl.ds(g * RPH, RPH)], sos[pb])
    out_cps[nchunks - 2].wait()
    out_cps[nchunks - 1].wait()


def _mems_sc(mmT, maskT, *, RPH):
    T, B, H, W, C = mmT.shape
    body = functools.partial(_mems_sc_body, T=T, B=B, H=H, W=W, C=C, RPH=RPH)
    return pl.kernel(
        body,
        out_type=jax.ShapeDtypeStruct((T, B, H, W, C), jnp.float32),
        mesh=_sc_mesh(),
        scratch_types=[
            pltpu.VMEM((H, W), jnp.float32),
            pltpu.VMEM((RPH, W, C), jnp.float32),
            pltpu.VMEM((RPH, W, C), jnp.float32),
            pltpu.VMEM((RPH, W, C), jnp.float32),
            pltpu.VMEM((RPH, W, C), jnp.float32),
            pltpu.SemaphoreType.DMA,
            pltpu.SemaphoreType.DMA,
            pltpu.SemaphoreType.DMA,
            pltpu.SemaphoreType.DMA,
        ],
        compiler_params=pltpu.CompilerParams(needs_layout_passes=False),
    )(mmT, maskT)


def _spikes_tc_body(mask_ref, sp_ref, os_ref):
    os_ref[...] = sp_ref[...] * mask_ref[...][None, :, :, :, None]


def _spikes_tc(spT, maskT, *, BH):
    T, B, H, W, C = spT.shape
    return pl.pallas_call(
        _spikes_tc_body,
        grid=(T, H // BH),
        in_specs=[
            pl.BlockSpec((B, BH, W), lambda i, j: (0, j, 0)),
            pl.BlockSpec((1, B, BH, W, C), lambda i, j: (i, 0, j, 0, 0)),
        ],
        out_specs=pl.BlockSpec((1, B, BH, W, C), lambda i, j: (i, 0, j, 0, 0)),
        out_shape=jax.ShapeDtypeStruct((T, B, H, W, C), jnp.float32),
    )(maskT, spT)


def kernel(spikes, mems, importance, training=0, target_override=None):
    target = target_override if target_override is not None else _TARGET_RATE
    T, B, C, H, W = spikes.shape
    HW = H * W
    k = max(1, int(target * HW))

    imp2d = importance.reshape(B, HW)
    u = jnp.clip(
        jax.random.uniform(jax.random.key(42), (B, 1, H, W), dtype=jnp.float32),
        1e-07, 1.0 - 1e-07).reshape(B, HW)
    g2_2d = -jnp.log(u) - 1e-07
    tr16 = jnp.broadcast_to(jnp.asarray(training, jnp.int32), (_L,))

    mask2d = _mask_sc(imp2d, g2_2d, tr16, k=k, B=B, HW=HW)
    maskT = mask2d.reshape(B, H, W)

    spT = jnp.transpose(spikes, (0, 1, 3, 4, 2))
    mmT = jnp.transpose(mems, (0, 1, 3, 4, 2))

    os_ = _spikes_tc(spT, maskT, BH=8)
    om_ = _mems_sc(mmT, maskT, RPH=2)

    return (jnp.transpose(os_, (0, 1, 4, 2, 3)),
            jnp.transpose(om_, (0, 1, 4, 2, 3)),
            mask2d.reshape(B, 1, H, W))


# speedup vs baseline: 1.4319x; 1.3833x over previous
import functools

import jax
import jax.numpy as jnp
from jax import lax
from jax.experimental import pallas as pl
from jax.experimental.pallas import tpu as pltpu
from jax.experimental.pallas import tpu_sc as plsc

_TARGET_RATE = 0.75
_TEMPERATURE = 0.5

_NC, _NS, _L = 2, 16, 16


def _sc_mesh():
    return plsc.VectorSubcoreMesh(
        core_axis_name="c", subcore_axis_name="s",
        num_cores=_NC, num_subcores=_NS)


def _mask_sc_body(imp_hbm, g2_hbm, tr_hbm, mask_hbm,
                  imp_v, g2_v, tr_v, out_v, hist, *, k, B, HW):
    nch = HW // _L
    w = lax.axis_index("s") * _NC + lax.axis_index("c")

    @pl.when(w < B)
    def _():
        pltpu.sync_copy(imp_hbm.at[w], imp_v)
        pltpu.sync_copy(g2_hbm.at[w], g2_v)
        pltpu.sync_copy(tr_hbm, tr_v)

        lanes = lax.iota(jnp.int32, _L)
        ones = jnp.ones((_L,), jnp.int32)
        zeros16 = jnp.zeros((_L,), jnp.int32)

        prefix = jnp.int32(0)
        k_rem = jnp.int32(k)
        for p in range(8):
            s = 28 - 4 * p
            sh = min(s + 4, 31)
            for z in range(_L):
                hist[pl.ds(z * _L, _L)] = zeros16

            def scan_chunk(c, acc, _s=s, _sh=sh, _prefix=prefix):
                for uu in range(4):
                    v = imp_v[pl.ds((c * 4 + uu) * _L, _L)]
                    bits = lax.bitcast_convert_type(v, jnp.int32)
                    match = lax.shift_right_logical(bits, _sh) == _prefix
                    digit = lax.shift_right_logical(bits, _s) & 15
                    plsc.addupdate_scatter(hist, [lanes * _L + digit], ones,
                                           mask=match)
                return acc

            lax.fori_loop(0, nch // 4, scan_chunk, jnp.int32(0))

            cnt = jnp.zeros((_L,), jnp.int32)
            for z in range(_L):
                cnt = cnt + hist[pl.ds(z * _L, _L)]
            rev = lax.rev(cnt, (0,))
            csrev = plsc.cumsum(rev)
            sfx_ex = csrev - rev
            is_sel = (csrev >= k_rem) & (sfx_ex < k_rem)
            d_star = jnp.sum(jnp.where(is_sel, 15 - lanes, 0))
            cnt_sel = jnp.sum(jnp.where(is_sel, rev, 0))
            k_rem = k_rem - jnp.sum(jnp.where(is_sel, sfx_ex, 0))
            prefix = lax.shift_left(prefix, 4) | d_star

        m = prefix
        extra = k_rem

        sel_train = tr_v[...] != 0

        @pl.when(cnt_sel == extra)
        def _no_excess_ties():
            def fin_fast(c, acc):
                for uu in range(4):
                    off = pl.ds((c * 4 + uu) * _L, _L)
                    v = imp_v[off]
                    bits = lax.bitcast_convert_type(v, jnp.int32)
                    take = bits >= m
                    train = v > g2_v[off] * (1.0 - v + 1e-07)
                    fin = jnp.where(sel_train, train, take)
                    out_v[off] = jnp.where(fin, 1.0, 0.0)
                return acc

            lax.fori_loop(0, nch // 4, fin_fast, jnp.int32(0))

        @pl.when(cnt_sel != extra)
        def _excess_ties():
            def fin_chunk(c, run_eq):
                v = imp_v[pl.ds(c * _L, _L)]
                bits = lax.bitcast_convert_type(v, jnp.int32)
                gt = bits > m
                eq = bits == m
                eqi = eq.astype(jnp.int32)
                ceq = plsc.cumsum(eqi) + run_eq
                take = gt | (eq & (ceq <= extra))
                train = v > g2_v[pl.ds(c * _L, _L)] * (1.0 - v + 1e-07)
                fin = jnp.where(sel_train, train, take)
                out_v[pl.ds(c * _L, _L)] = jnp.where(fin, 1.0, 0.0)
                return run_eq + jnp.sum(eqi)

            lax.fori_loop(0, nch, fin_chunk, jnp.int32(0))

        pltpu.sync_copy(out_v, mask_hbm.at[w])


def _mask_sc(imp2d, g2_2d, tr16, *, k, B, HW):
    body = functools.partial(_mask_sc_body, k=k, B=B, HW=HW)
    return pl.kernel(
        body,
        out_type=jax.ShapeDtypeStruct((B, HW), jnp.float32),
        mesh=_sc_mesh(),
        scratch_types=[
            pltpu.VMEM((HW,), jnp.float32),
            pltpu.VMEM((HW,), jnp.float32),
            pltpu.VMEM((_L,), jnp.int32),
            pltpu.VMEM((HW,), jnp.float32),
            pltpu.VMEM((_L * _L,), jnp.int32),
        ],
        compiler_params=pltpu.CompilerParams(needs_layout_passes=False),
    )(imp2d, g2_2d, tr16)


def _mems_sc_body(mm_hbm, mask_hbm, out_hbm,
                  mv_, bin0, bin1, bout0, bout1,
                  si0, si1, so0, so1, *, T, B, H, W, C, RPH):
    nchunks = H // RPH
    nc16 = C // _L
    wk = lax.axis_index("s") * _NC + lax.axis_index("c")
    t = lax.div(wk, B)
    b = lax.rem(wk, B)
    pltpu.sync_copy(mask_hbm.at[b], mv_)

    bins = (bin0, bin1)
    bouts = (bout0, bout1)
    sis = (si0, si1)
    sos = (so0, so1)

    in_cps = [None] * nchunks
    out_cps = [None] * nchunks
    in_cps[0] = pltpu.async_copy(mm_hbm.at[t, b, pl.ds(0, RPH)], bins[0],
                                 sis[0])
    for g in range(nchunks):
        pb = g % 2
        if g + 1 < nchunks:
            in_cps[g + 1] = pltpu.async_copy(
                mm_hbm.at[t, b, pl.ds((g + 1) * RPH, RPH)],
                bins[(g + 1) % 2], sis[(g + 1) % 2])
        in_cps[g].wait()
        if g >= 2:
            out_cps[g - 2].wait()
        bi, bo = bins[pb], bouts[pb]

        def mul_w(wi, acc, _bi=bi, _bo=bo, _g=g):
            iw = jnp.full((_L,), wi, jnp.int32)
            for r in range(RPH):
                ir = jnp.full((_L,), _g * RPH + r, jnp.int32)
                mvec = plsc.load_gather(mv_, [ir, iw])
                for c6 in range(nc16):
                    off = pl.ds(c6 * _L, _L)
                    _bo[r, wi, off] = _bi[r, wi, off] * mvec
            return acc

        lax.fori_loop(0, W, mul_w, jnp.int32(0))
        out_cps[g] = pltpu.async_copy(
            bo, out_hbm.at[t, b, pl.ds(g * RPH, RPH)], sos[pb])
    out_cps[nchunks - 2].wait()
    out_cps[nchunks - 1].wait()


def _mems_sc(mmT, maskT, *, RPH):
    T, B, H, W, C = mmT.shape
    body = functools.partial(_mems_sc_body, T=T, B=B, H=H, W=W, C=C, RPH=RPH)
    return pl.kernel(
        body,
        out_type=jax.ShapeDtypeStruct((T, B, H, W, C), jnp.float32),
        mesh=_sc_mesh(),
        scratch_types=[
            pltpu.VMEM((H, W), jnp.float32),
            pltpu.VMEM((RPH, W, C), jnp.float32),
            pltpu.VMEM((RPH, W, C), jnp.float32),
            pltpu.VMEM((RPH, W, C), jnp.float32),
            pltpu.VMEM((RPH, W, C), jnp.float32),
            pltpu.SemaphoreType.DMA,
            pltpu.SemaphoreType.DMA,
            pltpu.SemaphoreType.DMA,
            pltpu.SemaphoreType.DMA,
        ],
        compiler_params=pltpu.CompilerParams(needs_layout_passes=False),
    )(mmT, maskT)


def _spikes_tc_body(mask_ref, sp_ref, os_ref):
    os_ref[...] = sp_ref[...] * mask_ref[...][None, :, :, :, None]


def _spikes_tc(spT, maskT, *, BH):
    T, B, H, W, C = spT.shape
    return pl.pallas_call(
        _spikes_tc_body,
        grid=(T, H // BH),
        in_specs=[
            pl.BlockSpec((B, BH, W), lambda i, j: (0, j, 0)),
            pl.BlockSpec((1, B, BH, W, C), lambda i, j: (i, 0, j, 0, 0)),
        ],
        out_specs=pl.BlockSpec((1, B, BH, W, C), lambda i, j: (i, 0, j, 0, 0)),
        out_shape=jax.ShapeDtypeStruct((T, B, H, W, C), jnp.float32),
    )(maskT, spT)


def kernel(spikes, mems, importance, training=0, target_override=None):
    target = target_override if target_override is not None else _TARGET_RATE
    T, B, C, H, W = spikes.shape
    HW = H * W
    k = max(1, int(target * HW))

    imp2d = importance.reshape(B, HW)
    u = jnp.clip(
        jax.random.uniform(jax.random.key(42), (B, 1, H, W), dtype=jnp.float32),
        1e-07, 1.0 - 1e-07).reshape(B, HW)
    g2_2d = -jnp.log(u) - 1e-07
    tr16 = jnp.broadcast_to(jnp.asarray(training, jnp.int32), (_L,))

    mask2d = _mask_sc(imp2d, g2_2d, tr16, k=k, B=B, HW=HW)
    maskT = mask2d.reshape(B, H, W)

    spT = jnp.transpose(spikes, (0, 1, 3, 4, 2))
    mmT = jnp.transpose(mems, (0, 1, 3, 4, 2))

    os_ = _spikes_tc(spT, maskT, BH=8)
    om_ = _mems_sc(mmT, maskT, RPH=2)

    return (jnp.transpose(os_, (0, 1, 4, 2, 3)),
            jnp.transpose(om_, (0, 1, 4, 2, 3)),
            mask2d.reshape(B, 1, H, W))


# speedup vs baseline: 1.4532x; 1.0148x over previous
import functools

import jax
import jax.numpy as jnp
from jax import lax
from jax.experimental import pallas as pl
from jax.experimental.pallas import tpu as pltpu
from jax.experimental.pallas import tpu_sc as plsc

_TARGET_RATE = 0.75
_TEMPERATURE = 0.5

_NC, _NS, _L = 2, 16, 16


def _sc_mesh():
    return plsc.VectorSubcoreMesh(
        core_axis_name="c", subcore_axis_name="s",
        num_cores=_NC, num_subcores=_NS)


def _mask_sc_body(imp_hbm, g2_hbm, tr_hbm, mask_hbm,
                  imp_v, g2_v, tr_v, out_v, hist, *, k, B, HW):
    nch = HW // _L
    w = lax.axis_index("s") * _NC + lax.axis_index("c")

    @pl.when(w < B)
    def _():
        pltpu.sync_copy(imp_hbm.at[w], imp_v)
        pltpu.sync_copy(g2_hbm.at[w], g2_v)
        pltpu.sync_copy(tr_hbm, tr_v)

        lanes = lax.iota(jnp.int32, _L)
        ones = jnp.ones((_L,), jnp.int32)
        zeros16 = jnp.zeros((_L,), jnp.int32)

        nbin = 256
        prefix = jnp.int32(0)
        k_rem = jnp.int32(k)
        cnt_sel = jnp.int32(0)
        for p in range(4):
            s = 24 - 8 * p
            sh = min(s + 8, 31)
            for z in range(_L * nbin // _L):
                hist[pl.ds(z * _L, _L)] = zeros16

            def scan_chunk(c, acc, _s=s, _sh=sh, _prefix=prefix):
                for uu in range(4):
                    v = imp_v[pl.ds((c * 4 + uu) * _L, _L)]
                    bits = lax.bitcast_convert_type(v, jnp.int32)
                    match = lax.shift_right_logical(bits, _sh) == _prefix
                    digit = lax.shift_right_logical(bits, _s) & (nbin - 1)
                    plsc.addupdate_scatter(hist, [lanes * nbin + digit],
                                           ones, mask=match)
                return acc

            lax.fori_loop(0, nch // 4, scan_chunk, jnp.int32(0))

            running = jnp.int32(0)
            d_star = jnp.int32(0)
            taken = jnp.int32(0)
            csel = jnp.int32(0)
            for g in range(nbin // _L - 1, -1, -1):
                cnt = jnp.zeros((_L,), jnp.int32)
                for z in range(_L):
                    cnt = cnt + hist[pl.ds(z * nbin + g * _L, _L)]
                rev = lax.rev(cnt, (0,))
                csrev = plsc.cumsum(rev)
                sfx_in = running + csrev
                sfx_ex = sfx_in - rev
                is_sel = (sfx_in >= k_rem) & (sfx_ex < k_rem)
                d_star = d_star + jnp.sum(
                    jnp.where(is_sel, g * _L + 15 - lanes, 0))
                taken = taken + jnp.sum(jnp.where(is_sel, sfx_ex, 0))
                csel = csel + jnp.sum(jnp.where(is_sel, rev, 0))
                running = running + jnp.sum(cnt)
            cnt_sel = csel
            k_rem = k_rem - taken
            prefix = lax.shift_left(prefix, 8) | d_star

        m = prefix
        extra = k_rem

        sel_train = tr_v[...] != 0

        @pl.when(cnt_sel == extra)
        def _no_excess_ties():
            def fin_fast(c, acc):
                for uu in range(4):
                    off = pl.ds((c * 4 + uu) * _L, _L)
                    v = imp_v[off]
                    bits = lax.bitcast_convert_type(v, jnp.int32)
                    take = bits >= m
                    train = v > g2_v[off] * (1.0 - v + 1e-07)
                    fin = jnp.where(sel_train, train, take)
                    out_v[off] = jnp.where(fin, 1.0, 0.0)
                return acc

            lax.fori_loop(0, nch // 4, fin_fast, jnp.int32(0))

        @pl.when(cnt_sel != extra)
        def _excess_ties():
            def fin_chunk(c, run_eq):
                v = imp_v[pl.ds(c * _L, _L)]
                bits = lax.bitcast_convert_type(v, jnp.int32)
                gt = bits > m
                eq = bits == m
                eqi = eq.astype(jnp.int32)
                ceq = plsc.cumsum(eqi) + run_eq
                take = gt | (eq & (ceq <= extra))
                train = v > g2_v[pl.ds(c * _L, _L)] * (1.0 - v + 1e-07)
                fin = jnp.where(sel_train, train, take)
                out_v[pl.ds(c * _L, _L)] = jnp.where(fin, 1.0, 0.0)
                return run_eq + jnp.sum(eqi)

            lax.fori_loop(0, nch, fin_chunk, jnp.int32(0))

        pltpu.sync_copy(out_v, mask_hbm.at[w])


def _mask_sc(imp2d, g2_2d, tr16, *, k, B, HW):
    body = functools.partial(_mask_sc_body, k=k, B=B, HW=HW)
    return pl.kernel(
        body,
        out_type=jax.ShapeDtypeStruct((B, HW), jnp.float32),
        mesh=_sc_mesh(),
        scratch_types=[
            pltpu.VMEM((HW,), jnp.float32),
            pltpu.VMEM((HW,), jnp.float32),
            pltpu.VMEM((_L,), jnp.int32),
            pltpu.VMEM((HW,), jnp.float32),
            pltpu.VMEM((_L * 256,), jnp.int32),
        ],
        compiler_params=pltpu.CompilerParams(needs_layout_passes=False),
    )(imp2d, g2_2d, tr16)


def _mems_sc_body(mm_hbm, mask_hbm, out_hbm,
                  mv_, bin0, bin1, bout0, bout1,
                  si0, si1, so0, so1, *, T, B, H, W, C, RPH):
    nchunks = H // RPH
    nc16 = C // _L
    wk = lax.axis_index("s") * _NC + lax.axis_index("c")
    t = lax.div(wk, B)
    b = lax.rem(wk, B)
    pltpu.sync_copy(mask_hbm.at[b], mv_)

    bins = (bin0, bin1)
    bouts = (bout0, bout1)
    sis = (si0, si1)
    sos = (so0, so1)

    in_cps = [None] * nchunks
    out_cps = [None] * nchunks
    in_cps[0] = pltpu.async_copy(mm_hbm.at[t, b, pl.ds(0, RPH)], bins[0],
                                 sis[0])
    for g in range(nchunks):
        pb = g % 2
        if g + 1 < nchunks:
            in_cps[g + 1] = pltpu.async_copy(
                mm_hbm.at[t, b, pl.ds((g + 1) * RPH, RPH)],
                bins[(g + 1) % 2], sis[(g + 1) % 2])
        in_cps[g].wait()
        if g >= 2:
            out_cps[g - 2].wait()
        bi, bo = bins[pb], bouts[pb]

        def mul_w(wi, acc, _bi=bi, _bo=bo, _g=g):
            iw = jnp.full((_L,), wi, jnp.int32)
            for r in range(RPH):
                ir = jnp.full((_L,), _g * RPH + r, jnp.int32)
                mvec = plsc.load_gather(mv_, [ir, iw])
                for c6 in range(nc16):
                    off = pl.ds(c6 * _L, _L)
                    _bo[r, wi, off] = _bi[r, wi, off] * mvec
            return acc

        lax.fori_loop(0, W, mul_w, jnp.int32(0))
        out_cps[g] = pltpu.async_copy(
            bo, out_hbm.at[t, b, pl.ds(g * RPH, RPH)], sos[pb])
    out_cps[nchunks - 2].wait()
    out_cps[nchunks - 1].wait()


def _mems_sc(mmT, maskT, *, RPH):
    T, B, H, W, C = mmT.shape
    body = functools.partial(_mems_sc_body, T=T, B=B, H=H, W=W, C=C, RPH=RPH)
    return pl.kernel(
        body,
        out_type=jax.ShapeDtypeStruct((T, B, H, W, C), jnp.float32),
        mesh=_sc_mesh(),
        scratch_types=[
            pltpu.VMEM((H, W), jnp.float32),
            pltpu.VMEM((RPH, W, C), jnp.float32),
            pltpu.VMEM((RPH, W, C), jnp.float32),
            pltpu.VMEM((RPH, W, C), jnp.float32),
            pltpu.VMEM((RPH, W, C), jnp.float32),
            pltpu.SemaphoreType.DMA,
            pltpu.SemaphoreType.DMA,
            pltpu.SemaphoreType.DMA,
            pltpu.SemaphoreType.DMA,
        ],
        compiler_params=pltpu.CompilerParams(needs_layout_passes=False),
    )(mmT, maskT)


def _spikes_tc_body(mask_ref, sp_ref, os_ref):
    os_ref[...] = sp_ref[...] * mask_ref[...][None, :, :, :, None]


def _spikes_tc(spT, maskT, *, BH):
    T, B, H, W, C = spT.shape
    return pl.pallas_call(
        _spikes_tc_body,
        grid=(T, H // BH),
        in_specs=[
            pl.BlockSpec((B, BH, W), lambda i, j: (0, j, 0)),
            pl.BlockSpec((1, B, BH, W, C), lambda i, j: (i, 0, j, 0, 0)),
        ],
        out_specs=pl.BlockSpec((1, B, BH, W, C), lambda i, j: (i, 0, j, 0, 0)),
        out_shape=jax.ShapeDtypeStruct((T, B, H, W, C), jnp.float32),
    )(maskT, spT)


def kernel(spikes, mems, importance, training=0, target_override=None):
    target = target_override if target_override is not None else _TARGET_RATE
    T, B, C, H, W = spikes.shape
    HW = H * W
    k = max(1, int(target * HW))

    imp2d = importance.reshape(B, HW)
    u = jnp.clip(
        jax.random.uniform(jax.random.key(42), (B, 1, H, W), dtype=jnp.float32),
        1e-07, 1.0 - 1e-07).reshape(B, HW)
    g2_2d = -jnp.log(u) - 1e-07
    tr16 = jnp.broadcast_to(jnp.asarray(training, jnp.int32), (_L,))

    mask2d = _mask_sc(imp2d, g2_2d, tr16, k=k, B=B, HW=HW)
    maskT = mask2d.reshape(B, H, W)

    spT = jnp.transpose(spikes, (0, 1, 3, 4, 2))
    mmT = jnp.transpose(mems, (0, 1, 3, 4, 2))

    os_ = _spikes_tc(spT, maskT, BH=8)
    om_ = _mems_sc(mmT, maskT, RPH=2)

    return (jnp.transpose(os_, (0, 1, 4, 2, 3)),
            jnp.transpose(om_, (0, 1, 4, 2, 3)),
            mask2d.reshape(B, 1, H, W))


# speedup vs baseline: 1.5053x; 1.0358x over previous
import functools

import jax
import jax.numpy as jnp
from jax import lax
from jax.experimental import pallas as pl
from jax.experimental.pallas import tpu as pltpu
from jax.experimental.pallas import tpu_sc as plsc

_TARGET_RATE = 0.75
_TEMPERATURE = 0.5

_NC, _NS, _L = 2, 16, 16


def _sc_mesh():
    return plsc.VectorSubcoreMesh(
        core_axis_name="c", subcore_axis_name="s",
        num_cores=_NC, num_subcores=_NS)


def _mask_sc_body(imp_hbm, g2_hbm, tr_hbm, mask_hbm,
                  imp_v, g2_v, tr_v, out_v, hist, *, k, B, HW):
    nch = HW // _L
    w = lax.axis_index("s") * _NC + lax.axis_index("c")

    @pl.when(w < B)
    def _():
        pltpu.sync_copy(imp_hbm.at[w], imp_v)
        pltpu.sync_copy(g2_hbm.at[w], g2_v)
        pltpu.sync_copy(tr_hbm, tr_v)

        lanes = lax.iota(jnp.int32, _L)
        ones = jnp.ones((_L,), jnp.int32)
        zeros16 = jnp.zeros((_L,), jnp.int32)

        nbin = 256
        prefix = jnp.int32(0)
        k_rem = jnp.int32(k)
        cnt_sel = jnp.int32(0)
        for p in range(4):
            s = 24 - 8 * p
            sh = min(s + 8, 31)
            for z in range(_L * nbin // _L):
                hist[pl.ds(z * _L, _L)] = zeros16

            def scan_chunk(c, acc, _s=s, _sh=sh, _prefix=prefix):
                for uu in range(4):
                    v = imp_v[pl.ds((c * 4 + uu) * _L, _L)]
                    bits = lax.bitcast_convert_type(v, jnp.int32)
                    match = lax.shift_right_logical(bits, _sh) == _prefix
                    digit = lax.shift_right_logical(bits, _s) & (nbin - 1)
                    plsc.addupdate_scatter(hist, [lanes * nbin + digit],
                                           ones, mask=match)
                return acc

            lax.fori_loop(0, nch // 4, scan_chunk, jnp.int32(0))

            running = jnp.int32(0)
            d_star = jnp.int32(0)
            taken = jnp.int32(0)
            csel = jnp.int32(0)
            for g in range(nbin // _L - 1, -1, -1):
                cnt = jnp.zeros((_L,), jnp.int32)
                for z in range(_L):
                    cnt = cnt + hist[pl.ds(z * nbin + g * _L, _L)]
                rev = lax.rev(cnt, (0,))
                csrev = plsc.cumsum(rev)
                sfx_in = running + csrev
                sfx_ex = sfx_in - rev
                is_sel = (sfx_in >= k_rem) & (sfx_ex < k_rem)
                d_star = d_star + jnp.sum(
                    jnp.where(is_sel, g * _L + 15 - lanes, 0))
                taken = taken + jnp.sum(jnp.where(is_sel, sfx_ex, 0))
                csel = csel + jnp.sum(jnp.where(is_sel, rev, 0))
                running = running + jnp.sum(cnt)
            cnt_sel = csel
            k_rem = k_rem - taken
            prefix = lax.shift_left(prefix, 8) | d_star

        m = prefix
        extra = k_rem

        sel_train = tr_v[...] != 0

        @pl.when(cnt_sel == extra)
        def _no_excess_ties():
            def fin_fast(c, acc):
                for uu in range(4):
                    off = pl.ds((c * 4 + uu) * _L, _L)
                    v = imp_v[off]
                    bits = lax.bitcast_convert_type(v, jnp.int32)
                    take = bits >= m
                    train = v > g2_v[off] * (1.0 - v + 1e-07)
                    fin = jnp.where(sel_train, train, take)
                    out_v[off] = jnp.where(fin, 1.0, 0.0)
                return acc

            lax.fori_loop(0, nch // 4, fin_fast, jnp.int32(0))

        @pl.when(cnt_sel != extra)
        def _excess_ties():
            def fin_chunk(c, run_eq):
                v = imp_v[pl.ds(c * _L, _L)]
                bits = lax.bitcast_convert_type(v, jnp.int32)
                gt = bits > m
                eq = bits == m
                eqi = eq.astype(jnp.int32)
                ceq = plsc.cumsum(eqi) + run_eq
                take = gt | (eq & (ceq <= extra))
                train = v > g2_v[pl.ds(c * _L, _L)] * (1.0 - v + 1e-07)
                fin = jnp.where(sel_train, train, take)
                out_v[pl.ds(c * _L, _L)] = jnp.where(fin, 1.0, 0.0)
                return run_eq + jnp.sum(eqi)

            lax.fori_loop(0, nch, fin_chunk, jnp.int32(0))

        pltpu.sync_copy(out_v, mask_hbm.at[w])


def _mask_sc(imp2d, g2_2d, tr16, *, k, B, HW):
    body = functools.partial(_mask_sc_body, k=k, B=B, HW=HW)
    return pl.kernel(
        body,
        out_type=jax.ShapeDtypeStruct((B, HW), jnp.float32),
        mesh=_sc_mesh(),
        scratch_types=[
            pltpu.VMEM((HW,), jnp.float32),
            pltpu.VMEM((HW,), jnp.float32),
            pltpu.VMEM((_L,), jnp.int32),
            pltpu.VMEM((HW,), jnp.float32),
            pltpu.VMEM((_L * 256,), jnp.int32),
        ],
        compiler_params=pltpu.CompilerParams(needs_layout_passes=False),
    )(imp2d, g2_2d, tr16)


def _mems_sc_body(mm_hbm, mask_hbm, out_hbm,
                  mv_, bin0, bin1, bout0, bout1,
                  si0, si1, so0, so1, *, T, B, H, W, C, RPH):
    nchunks = H // RPH
    nc16 = C // _L
    wk = lax.axis_index("s") * _NC + lax.axis_index("c")
    t = lax.div(wk, B)
    b = lax.rem(wk, B)
    pltpu.sync_copy(mask_hbm.at[b], mv_)

    bins = (bin0, bin1)
    bouts = (bout0, bout1)
    sis = (si0, si1)
    sos = (so0, so1)

    in_cps = [None] * nchunks
    out_cps = [None] * nchunks
    in_cps[0] = pltpu.async_copy(mm_hbm.at[t, b, pl.ds(0, RPH)], bins[0],
                                 sis[0])
    for g in range(nchunks):
        pb = g % 2
        if g + 1 < nchunks:
            in_cps[g + 1] = pltpu.async_copy(
                mm_hbm.at[t, b, pl.ds((g + 1) * RPH, RPH)],
                bins[(g + 1) % 2], sis[(g + 1) % 2])
        in_cps[g].wait()
        if g >= 2:
            out_cps[g - 2].wait()
        bi, bo = bins[pb], bouts[pb]

        def mul_w(wi, acc, _bi=bi, _bo=bo, _g=g):
            iw = jnp.full((_L,), wi, jnp.int32)
            for r in range(RPH):
                ir = jnp.full((_L,), _g * RPH + r, jnp.int32)
                mvec = plsc.load_gather(mv_, [ir, iw])
                for c6 in range(nc16):
                    off = pl.ds(c6 * _L, _L)
                    _bo[r, wi, off] = _bi[r, wi, off] * mvec
            return acc

        lax.fori_loop(0, W, mul_w, jnp.int32(0))
        out_cps[g] = pltpu.async_copy(
            bo, out_hbm.at[t, b, pl.ds(g * RPH, RPH)], sos[pb])
    out_cps[nchunks - 2].wait()
    out_cps[nchunks - 1].wait()


def _mems_sc(mmT, maskT, *, RPH):
    T, B, H, W, C = mmT.shape
    body = functools.partial(_mems_sc_body, T=T, B=B, H=H, W=W, C=C, RPH=RPH)
    return pl.kernel(
        body,
        out_type=jax.ShapeDtypeStruct((T, B, H, W, C), jnp.float32),
        mesh=_sc_mesh(),
        scratch_types=[
            pltpu.VMEM((H, W), jnp.float32),
            pltpu.VMEM((RPH, W, C), jnp.float32),
            pltpu.VMEM((RPH, W, C), jnp.float32),
            pltpu.VMEM((RPH, W, C), jnp.float32),
            pltpu.VMEM((RPH, W, C), jnp.float32),
            pltpu.SemaphoreType.DMA,
            pltpu.SemaphoreType.DMA,
            pltpu.SemaphoreType.DMA,
            pltpu.SemaphoreType.DMA,
        ],
        compiler_params=pltpu.CompilerParams(needs_layout_passes=False),
    )(mmT, maskT)


def _spikes_tc_body(mask_ref, sp_ref, os_ref):
    os_ref[...] = sp_ref[...] * mask_ref[...][None, :, :, :, None]


def _spikes_tc(spT, maskT, *, BH):
    T, B, H, W, C = spT.shape
    return pl.pallas_call(
        _spikes_tc_body,
        grid=(T, H // BH),
        in_specs=[
            pl.BlockSpec((B, BH, W), lambda i, j: (0, j, 0)),
            pl.BlockSpec((1, B, BH, W, C), lambda i, j: (i, 0, j, 0, 0)),
        ],
        out_specs=pl.BlockSpec((1, B, BH, W, C), lambda i, j: (i, 0, j, 0, 0)),
        out_shape=jax.ShapeDtypeStruct((T, B, H, W, C), jnp.float32),
    )(maskT, spT)


def kernel(spikes, mems, importance, training=0, target_override=None):
    target = target_override if target_override is not None else _TARGET_RATE
    T, B, C, H, W = spikes.shape
    HW = H * W
    k = max(1, int(target * HW))

    imp2d = importance.reshape(B, HW)
    u = jnp.clip(
        jax.random.uniform(jax.random.key(42), (B, 1, H, W), dtype=jnp.float32),
        1e-07, 1.0 - 1e-07).reshape(B, HW)
    g2_2d = -jnp.log(u) - 1e-07
    tr16 = jnp.broadcast_to(jnp.asarray(training, jnp.int32), (_L,))

    mask2d = _mask_sc(imp2d, g2_2d, tr16, k=k, B=B, HW=HW)
    maskT = mask2d.reshape(B, H, W)

    spT = jnp.transpose(spikes, (0, 1, 3, 4, 2))
    mmT = jnp.transpose(mems, (0, 1, 3, 4, 2))

    os_ = _spikes_tc(spT, maskT, BH=16)
    om_ = _mems_sc(mmT, maskT, RPH=2)

    return (jnp.transpose(os_, (0, 1, 4, 2, 3)),
            jnp.transpose(om_, (0, 1, 4, 2, 3)),
            mask2d.reshape(B, 1, H, W))
